# baseline (speedup 1.0000x reference)
import functools

import jax
import jax.numpy as jnp
from jax import lax
from jax.experimental import pallas as pl
from jax.experimental.pallas import tpu as pltpu
from jax.experimental.pallas import tpu_sc as plsc

NC = 2
NS = 16
NW = NC * NS

D = 64
LANES = 16
BCH = 128


def _sc_body(seq_len, idxT_hbm, pos_hbm, token_hbm, out_hbm,
             idx_v, buf, tbuf, pos_v, gsem, osem):
    w = lax.axis_index("s") * NC + lax.axis_index("c")
    b0 = w * BCH

    pltpu.sync_copy(pos_hbm, pos_v)
    iota = lax.iota(jnp.int32, LANES)

    def fire_gather(s, b):
        pltpu.sync_copy(idxT_hbm.at[s, pl.ds(b0, BCH)], idx_v[b])
        pltpu.async_copy(token_hbm.at[idx_v[b]], buf[b], gsem[b])

    def drain_g(b):
        pltpu.make_async_copy(token_hbm.at[pl.ds(0, BCH)], buf[b],
                              gsem[b]).wait()

    def drain_o(b):
        pltpu.make_async_copy(out_hbm.at[0, :, 0, :], tbuf[b],
                              osem[b]).wait()

    def transpose_add(s, b):
        def rd_body(rd, carry):
            for td in range(D // 8):
                d = td * 8 + rd
                ps = plsc.load_gather(
                    pos_v, [jnp.full((LANES,), s, jnp.int32),
                            jnp.full((LANES,), d, jnp.int32)])
                for k in range(BCH // LANES):
                    col = plsc.load_gather(
                        buf[b], [iota + (k * LANES),
                                 jnp.full((LANES,), d, jnp.int32)])
                    tbuf[b][td, pl.ds(rd * 128 + k * LANES, LANES)] = col + ps
            return carry
        lax.fori_loop(0, 8, rd_body, 0)

    fire_gather(0, 0)

    def pair_body(p, carry):
        for b in range(2):
            s = 2 * p + b

            @pl.when(s + 1 < seq_len)
            def _prefetch():
                fire_gather(s + 1, 1 - b)

            drain_g(b)

            @pl.when(s >= 2)
            def _wait_writeout():
                drain_o(b)

            transpose_add(s, b)
            pltpu.async_copy(tbuf[b], out_hbm.at[s, :, w, :], osem[b])
        return carry

    lax.fori_loop(0, seq_len // 2, pair_body, 0)
    drain_o(0)
    drain_o(1)


@functools.partial(jax.jit, static_argnums=(3, 4))
def _sc_embed(idxT, pos_table, token_table, batch, seq_len):
    n_tb = batch // BCH
    mesh = plsc.VectorSubcoreMesh(core_axis_name="c", subcore_axis_name="s",
                                  num_cores=NC, num_subcores=NS)
    body = functools.partial(_sc_body, seq_len)
    out = pl.kernel(
        body,
        out_type=jax.ShapeDtypeStruct((seq_len, D // 8, n_tb, 8 * BCH),
                                      jnp.float32),
        mesh=mesh,
        compiler_params=pltpu.CompilerParams(use_tc_tiling_on_sc=False,
                                             needs_layout_passes=False),
        scratch_types=[
            [pltpu.VMEM((BCH,), jnp.int32)] * 2,
            [pltpu.VMEM((BCH, D), jnp.float32)] * 2,
            [pltpu.VMEM((D // 8, 8 * BCH), jnp.float32)] * 2,
            pltpu.VMEM((seq_len, D), jnp.float32),
            [pltpu.SemaphoreType.DMA] * 2,
            [pltpu.SemaphoreType.DMA] * 2,
        ],
    )(idxT, pos_table, token_table)
    return out


def kernel(inputs, token_table, pos_table):
    batch, seq_len = inputs.shape
    assert batch == NW * BCH and seq_len % 2 == 0 and token_table.shape[1] == D
    idxT = inputs.T
    out4 = _sc_embed(idxT, pos_table, token_table, batch, seq_len)
    out5 = out4.reshape(seq_len, D // 8, batch // BCH, 8, BCH)
    return out5.transpose(2, 4, 0, 1, 3).reshape(batch, seq_len, D)


# speedup vs baseline: 1.2031x; 1.2031x over previous
import functools

import jax
import jax.numpy as jnp
from jax import lax
from jax.experimental import pallas as pl
from jax.experimental.pallas import tpu as pltpu
from jax.experimental.pallas import tpu_sc as plsc

NC = 2
NS = 16
NW = NC * NS

D = 64
LANES = 16
BCH = 128


def _sc_body(seq_len, idxT_hbm, pos_hbm, token_hbm, out_hbm,
             idx_v, buf, tbuf, pos_v, gsem, osem):
    w = lax.axis_index("s") * NC + lax.axis_index("c")
    b0 = w * BCH

    pltpu.sync_copy(pos_hbm, pos_v)
    pltpu.sync_copy(idxT_hbm.at[:, pl.ds(b0, BCH)], idx_v)
    iota = lax.iota(jnp.int32, LANES)
    row_vecs = [iota // 8 + 2 * c2 for c2 in range(D // LANES)]
    col_base = (iota % 8) * 128

    def fire_gather(s, b):
        pltpu.async_copy(token_hbm.at[idx_v.at[s]], buf[b], gsem[b])

    def drain_g(b):
        pltpu.make_async_copy(token_hbm.at[pl.ds(0, BCH)], buf[b],
                              gsem[b]).wait()

    def drain_o(b):
        pltpu.make_async_copy(out_hbm.at[0, :, 0, :], tbuf[b],
                              osem[b]).wait()

    def transpose_add(s, b):
        pv = [pos_v[s, pl.ds(c2 * LANES, LANES)] for c2 in range(D // LANES)]

        def c_body(c8, carry):
            for cu in range(8):
                c = c8 * 8 + cu
                col = col_base + c
                for c2 in range(D // LANES):
                    v = buf[b][c, pl.ds(c2 * LANES, LANES)] + pv[c2]
                    plsc.store_scatter(tbuf[b], [row_vecs[c2], col], v)
            return carry
        lax.fori_loop(0, BCH // 8, c_body, 0)

    fire_gather(0, 0)

    def pair_body(p, carry):
        for b in range(2):
            s = 2 * p + b

            @pl.when(s + 1 < seq_len)
            def _prefetch():
                fire_gather(s + 1, 1 - b)

            drain_g(b)

            @pl.when(s >= 2)
            def _wait_writeout():
                drain_o(b)

            transpose_add(s, b)
            pltpu.async_copy(tbuf[b], out_hbm.at[s, :, w, :], osem[b])
        return carry

    lax.fori_loop(0, seq_len // 2, pair_body, 0)
    drain_o(0)
    drain_o(1)


@functools.partial(jax.jit, static_argnums=(3, 4))
def _sc_embed(idxT, pos_table, token_table, batch, seq_len):
    n_tb = batch // BCH
    mesh = plsc.VectorSubcoreMesh(core_axis_name="c", subcore_axis_name="s",
                                  num_cores=NC, num_subcores=NS)
    body = functools.partial(_sc_body, seq_len)
    out = pl.kernel(
        body,
        out_type=jax.ShapeDtypeStruct((seq_len, D // 8, n_tb, 8 * BCH),
                                      jnp.float32),
        mesh=mesh,
        compiler_params=pltpu.CompilerParams(use_tc_tiling_on_sc=False,
                                             needs_layout_passes=False),
        scratch_types=[
            pltpu.VMEM((seq_len, BCH), jnp.int32),
            [pltpu.VMEM((BCH, D), jnp.float32)] * 2,
            [pltpu.VMEM((D // 8, 8 * BCH), jnp.float32)] * 2,
            pltpu.VMEM((seq_len, D), jnp.float32),
            [pltpu.SemaphoreType.DMA] * 2,
            [pltpu.SemaphoreType.DMA] * 2,
        ],
    )(idxT, pos_table, token_table)
    return out


def kernel(inputs, token_table, pos_table):
    batch, seq_len = inputs.shape
    assert batch == NW * BCH and seq_len % 2 == 0 and token_table.shape[1] == D
    idxT = inputs.T
    out4 = _sc_embed(idxT, pos_table, token_table, batch, seq_len)
    out5 = out4.reshape(seq_len, D // 8, batch // BCH, 8, BCH)
    return out5.transpose(2, 4, 0, 1, 3).reshape(batch, seq_len, D)


# speedup vs baseline: 1.5561x; 1.2934x over previous
import functools

import jax
import jax.numpy as jnp
from jax import lax
from jax.experimental import pallas as pl
from jax.experimental.pallas import tpu as pltpu
from jax.experimental.pallas import tpu_sc as plsc

NC = 2
NS = 16
NW = NC * NS

D = 64
LANES = 16
BCH = 128


def _sc_body(seq_len, idxT_hbm, pos_hbm, token_hbm, out_hbm,
             idx_v, buf, tbuf, pos_v, gsem, osem):
    w = lax.axis_index("s") * NC + lax.axis_index("c")
    b0 = w * BCH

    pltpu.sync_copy(pos_hbm, pos_v)
    pltpu.sync_copy(idxT_hbm.at[:, pl.ds(b0, BCH)], idx_v)
    iota = lax.iota(jnp.int32, LANES)
    row_vecs = [iota // 8 + 2 * c2 for c2 in range(D // LANES)]
    col_base = (iota % 8) * 128

    def fire_gather(s, b):
        pltpu.async_copy(token_hbm.at[idx_v.at[s]], buf[b], gsem[b])

    def drain_g(b):
        pltpu.make_async_copy(token_hbm.at[pl.ds(0, BCH)], buf[b],
                              gsem[b]).wait()

    def drain_o(b):
        pltpu.make_async_copy(out_hbm.at[0, :, 0, :], tbuf[b],
                              osem[b]).wait()

    def transpose_add(s, b):
        pv = [pos_v[s, pl.ds(c2 * LANES, LANES)] for c2 in range(D // LANES)]

        @plsc.parallel_loop(0, BCH, 8, unroll=2)
        def c_body(c8):
            for cu in range(8):
                c = c8 + cu
                col = col_base + c
                for c2 in range(D // LANES):
                    v = buf[b][c, pl.ds(c2 * LANES, LANES)] + pv[c2]
                    plsc.store_scatter(tbuf[b], [row_vecs[c2], col], v)

    fire_gather(0, 0)

    def pair_body(p, carry):
        for b in range(2):
            s = 2 * p + b

            @pl.when(s + 1 < seq_len)
            def _prefetch():
                fire_gather(s + 1, 1 - b)

            drain_g(b)

            @pl.when(s >= 2)
            def _wait_writeout():
                drain_o(b)

            transpose_add(s, b)
            pltpu.async_copy(tbuf[b], out_hbm.at[s, :, w, :], osem[b])
        return carry

    lax.fori_loop(0, seq_len // 2, pair_body, 0)
    drain_o(0)
    drain_o(1)


@functools.partial(jax.jit, static_argnums=(3, 4))
def _sc_embed(idxT, pos_table, token_table, batch, seq_len):
    n_tb = batch // BCH
    mesh = plsc.VectorSubcoreMesh(core_axis_name="c", subcore_axis_name="s",
                                  num_cores=NC, num_subcores=NS)
    body = functools.partial(_sc_body, seq_len)
    out = pl.kernel(
        body,
        out_type=jax.ShapeDtypeStruct((seq_len, D // 8, n_tb, 8 * BCH),
                                      jnp.float32),
        mesh=mesh,
        compiler_params=pltpu.CompilerParams(use_tc_tiling_on_sc=False,
                                             needs_layout_passes=False),
        scratch_types=[
            pltpu.VMEM((seq_len, BCH), jnp.int32),
            [pltpu.VMEM((BCH, D), jnp.float32)] * 2,
            [pltpu.VMEM((D // 8, 8 * BCH), jnp.float32)] * 2,
            pltpu.VMEM((seq_len, D), jnp.float32),
            [pltpu.SemaphoreType.DMA] * 2,
            [pltpu.SemaphoreType.DMA] * 2,
        ],
    )(idxT, pos_table, token_table)
    return out


def kernel(inputs, token_table, pos_table):
    batch, seq_len = inputs.shape
    assert batch == NW * BCH and seq_len % 2 == 0 and token_table.shape[1] == D
    idxT = inputs.T
    out4 = _sc_embed(idxT, pos_table, token_table, batch, seq_len)
    out5 = out4.reshape(seq_len, D // 8, batch // BCH, 8, BCH)
    return out5.transpose(2, 4, 0, 1, 3).reshape(batch, seq_len, D)


# speedup vs baseline: 1.5657x; 1.0062x over previous
import functools

import jax
import jax.numpy as jnp
from jax import lax
from jax.experimental import pallas as pl
from jax.experimental.pallas import tpu as pltpu
from jax.experimental.pallas import tpu_sc as plsc

NC = 2
NS = 16
NW = NC * NS

D = 64
LANES = 16
BCH = 128


def _sc_body(seq_len, idxT_hbm, pos_hbm, token_hbm, out_hbm,
             idx_v, buf, tbuf, pos_v, gsem, osem):
    w = lax.axis_index("s") * NC + lax.axis_index("c")
    b0 = w * BCH

    pltpu.sync_copy(pos_hbm, pos_v)
    pltpu.sync_copy(idxT_hbm.at[:, pl.ds(b0, BCH)], idx_v)
    iota = lax.iota(jnp.int32, LANES)
    zero_vec = iota * 0
    flat_base = [(iota // 8) * 1024 + (iota % 8) * 128 + 2048 * c2
                 for c2 in range(D // LANES)]

    def fire_gather(s, b):
        pltpu.async_copy(token_hbm.at[idx_v.at[s]], buf[b], gsem[b])

    def drain_g(b):
        pltpu.make_async_copy(token_hbm.at[pl.ds(0, BCH)], buf[b],
                              gsem[b]).wait()

    def drain_o(b):
        pltpu.make_async_copy(out_hbm.at[0, :, 0, :], tbuf[b],
                              osem[b]).wait()

    def transpose_add(s, b):
        pv = [pos_v[s, pl.ds(c2 * LANES, LANES)] for c2 in range(D // LANES)]

        @plsc.parallel_loop(0, BCH, 8, unroll=4)
        def c_body(c8):
            for cu in range(8):
                c = c8 + cu
                for c2 in range(D // LANES):
                    v = buf[b][c, pl.ds(c2 * LANES, LANES)] + pv[c2]
                    plsc.store_scatter(tbuf[b],
                                       [zero_vec, flat_base[c2] + c], v)

    fire_gather(0, 0)

    def pair_body(p, carry):
        for b in range(2):
            s = 2 * p + b

            @pl.when(s + 1 < seq_len)
            def _prefetch():
                fire_gather(s + 1, 1 - b)

            drain_g(b)

            @pl.when(s >= 2)
            def _wait_writeout():
                drain_o(b)

            transpose_add(s, b)
            pltpu.async_copy(tbuf[b], out_hbm.at[s, :, w, :], osem[b])
        return carry

    lax.fori_loop(0, seq_len // 2, pair_body, 0)
    drain_o(0)
    drain_o(1)


@functools.partial(jax.jit, static_argnums=(3, 4))
def _sc_embed(idxT, pos_table, token_table, batch, seq_len):
    n_tb = batch // BCH
    mesh = plsc.VectorSubcoreMesh(core_axis_name="c", subcore_axis_name="s",
                                  num_cores=NC, num_subcores=NS)
    body = functools.partial(_sc_body, seq_len)
    out = pl.kernel(
        body,
        out_type=jax.ShapeDtypeStruct((seq_len, D // 8, n_tb, 8 * BCH),
                                      jnp.float32),
        mesh=mesh,
        compiler_params=pltpu.CompilerParams(use_tc_tiling_on_sc=False,
                                             needs_layout_passes=False),
        scratch_types=[
            pltpu.VMEM((seq_len, BCH), jnp.int32),
            [pltpu.VMEM((BCH, D), jnp.float32)] * 2,
            [pltpu.VMEM((D // 8, 8 * BCH), jnp.float32)] * 2,
            pltpu.VMEM((seq_len, D), jnp.float32),
            [pltpu.SemaphoreType.DMA] * 2,
            [pltpu.SemaphoreType.DMA] * 2,
        ],
    )(idxT, pos_table, token_table)
    return out


def kernel(inputs, token_table, pos_table):
    batch, seq_len = inputs.shape
    assert batch == NW * BCH and seq_len % 2 == 0 and token_table.shape[1] == D
    idxT = inputs.T
    out4 = _sc_embed(idxT, pos_table, token_table, batch, seq_len)
    out5 = out4.reshape(seq_len, D // 8, batch // BCH, 8, BCH)
    return out5.transpose(2, 4, 0, 1, 3).reshape(batch, seq_len, D)


# speedup vs baseline: 2.2368x; 1.4287x over previous
import functools

import jax
import jax.numpy as jnp
from jax import lax
from jax.experimental import pallas as pl
from jax.experimental.pallas import tpu as pltpu
from jax.experimental.pallas import tpu_sc as plsc

NC = 2
NS = 16
NW = NC * NS

D = 64
LANES = 16
BCH = 128


def _sc_body(seq_len, idxT_hbm, pos_hbm, token_hbm, out_hbm,
             idx_v, buf, tbuf, pos_v, gsem, osem):
    w = lax.axis_index("s") * NC + lax.axis_index("c")
    b0 = w * BCH

    pltpu.sync_copy(pos_hbm, pos_v)
    pltpu.sync_copy(idxT_hbm.at[:, pl.ds(b0, BCH)], idx_v)
    iota = lax.iota(jnp.int32, LANES)
    zero_vec = iota * 0
    rowk64 = [(iota + LANES * k) * D for k in range(BCH // LANES)]
    rowk = [iota + LANES * k for k in range(BCH // LANES)]

    def fire_gather(s, b):
        pltpu.async_copy(token_hbm.at[idx_v.at[s]], buf[b], gsem[b])

    def drain_g(b):
        pltpu.make_async_copy(token_hbm.at[pl.ds(0, BCH)], buf[b],
                              gsem[b]).wait()

    def drain_o(b):
        pltpu.make_async_copy(out_hbm.at[0, :, 0, :], tbuf[b],
                              osem[b]).wait()

    def transpose_add(s, b):
        s64 = jnp.full((LANES,), s * D, jnp.int32)

        @plsc.parallel_loop(0, LANES, 1, unroll=2)
        def r_body(r):
            m16 = (iota + r) % LANES
            m8x128 = (m16 % 8) * 128
            tdrow = (m16 // 8) * 1024 + m8x128
            for c2 in range(D // LANES):
                dgoff = m16 + c2 * LANES
                ps = plsc.load_gather(pos_v, [zero_vec, dgoff + s64])
                base = tdrow + c2 * 2048
                for k in range(BCH // LANES):
                    v = plsc.load_gather(buf[b], [zero_vec, rowk64[k] + dgoff])
                    plsc.store_scatter(tbuf[b], [zero_vec, base + rowk[k]],
                                       v + ps)

    fire_gather(0, 0)

    def pair_body(p, carry):
        for b in range(2):
            s = 2 * p + b

            @pl.when(s + 1 < seq_len)
            def _prefetch():
                fire_gather(s + 1, 1 - b)

            drain_g(b)

            @pl.when(s >= 2)
            def _wait_writeout():
                drain_o(b)

            transpose_add(s, b)
            pltpu.async_copy(tbuf[b], out_hbm.at[s, :, w, :], osem[b])
        return carry

    lax.fori_loop(0, seq_len // 2, pair_body, 0)
    drain_o(0)
    drain_o(1)


@functools.partial(jax.jit, static_argnums=(3, 4))
def _sc_embed(idxT, pos_table, token_table, batch, seq_len):
    n_tb = batch // BCH
    mesh = plsc.VectorSubcoreMesh(core_axis_name="c", subcore_axis_name="s",
                                  num_cores=NC, num_subcores=NS)
    body = functools.partial(_sc_body, seq_len)
    out = pl.kernel(
        body,
        out_type=jax.ShapeDtypeStruct((seq_len, D // 8, n_tb, 8 * BCH),
                                      jnp.float32),
        mesh=mesh,
        compiler_params=pltpu.CompilerParams(use_tc_tiling_on_sc=False,
                                             needs_layout_passes=False),
        scratch_types=[
            pltpu.VMEM((seq_len, BCH), jnp.int32),
            [pltpu.VMEM((BCH, D), jnp.float32)] * 2,
            [pltpu.VMEM((D // 8, 8 * BCH), jnp.float32)] * 2,
            pltpu.VMEM((seq_len, D), jnp.float32),
            [pltpu.SemaphoreType.DMA] * 2,
            [pltpu.SemaphoreType.DMA] * 2,
        ],
    )(idxT, pos_table, token_table)
    return out


def kernel(inputs, token_table, pos_table):
    batch, seq_len = inputs.shape
    assert batch == NW * BCH and seq_len % 2 == 0 and token_table.shape[1] == D
    idxT = inputs.T
    out4 = _sc_embed(idxT, pos_table, token_table, batch, seq_len)
    out5 = out4.reshape(seq_len, D // 8, batch // BCH, 8, BCH)
    return out5.transpose(2, 4, 0, 1, 3).reshape(batch, seq_len, D)
